# baseline (speedup 1.0000x reference)
import functools

import jax
import jax.numpy as jnp
from jax import lax
from jax.experimental import pallas as pl
from jax.experimental.pallas import tpu as pltpu
from jax.experimental.pallas import tpu_sc as plsc

NC = 2
NS = 16
NW = NC * NS
W = 8
LANE = 16
BT = 128


@functools.lru_cache(maxsize=None)
def _emb_kernel(b, seq, d, vocab):
    nchunk = seq // W
    ngroup = (nchunk - 1) // 2
    te = d // 8
    mesh = plsc.VectorSubcoreMesh(core_axis_name="c", subcore_axis_name="s")

    @functools.partial(
        pl.kernel,
        mesh=mesh,
        compiler_params=pltpu.CompilerParams(use_tc_tiling_on_sc=False,
                                             needs_layout_passes=False),
        out_type=jax.ShapeDtypeStruct((seq, te, b // BT, 8, BT), jnp.float32),
        scratch_types=[
            pltpu.VMEM((2, W, BT), jnp.int32),
            pltpu.VMEM((2, W, BT, d), jnp.float32),
            pltpu.VMEM((2, te, 8, BT), jnp.float32),
            pltpu.VMEM((d, seq), jnp.float32),
            [pltpu.SemaphoreType.DMA] * 2,
            [pltpu.SemaphoreType.DMA] * 2,
            [pltpu.SemaphoreType.DMA] * 2,
        ],
    )
    def k(ids_t, tok_hbm, pos_t, out5, idx_v, rows_v, slab_v, pos_v,
          sem_i, sem_g, sem_o):
        wid = lax.axis_index("s") * NC + lax.axis_index("c")
        b0 = pl.multiple_of(wid * BT, BT)
        pltpu.sync_copy(pos_t.at[:, pl.ds(0, seq)], pos_v)
        riota = [lax.iota(jnp.int32, 16) + bb * LANE
                 for bb in range(BT // LANE)]

        def idx_copy(c, slot):
            s0 = pl.multiple_of(c * W, W)
            return pltpu.make_async_copy(
                ids_t.at[pl.ds(s0, W), pl.ds(b0, BT)],
                idx_v.at[slot], sem_i[slot])

        def gather(slot, sl):
            return pltpu.make_async_copy(
                tok_hbm.at[idx_v.at[slot, sl]], rows_v.at[slot, sl],
                sem_g[slot])

        def slab_out(c, sl):
            return pltpu.make_async_copy(
                slab_v.at[sl % 2],
                out5.at[c * W + sl, :, wid], sem_o[sl % 2])

        def compute(c, slot):
            for sl in range(W):
                if sl >= 2:
                    slab_out(c, sl - 2).wait()
                else:
                    @pl.when(c >= 1)
                    def _():
                        slab_out(c - 1, W + sl - 2).wait()

                def e_body(e, carry):
                    col = jnp.full((16,), e, jnp.int32)
                    pv = plsc.load_gather(
                        pos_v, [col, jnp.full((16,), c * W + sl, jnp.int32)])
                    for bb in range(BT // LANE):
                        v = plsc.load_gather(rows_v.at[slot, sl],
                                             [riota[bb], col])
                        slab_v[sl % 2, e // 8, lax.rem(e, 8),
                               pl.ds(bb * LANE, LANE)] = v + pv
                    return carry

                lax.fori_loop(0, d, e_body, 0)
                slab_out(c, sl).start()

        idx_copy(0, 0).start()
        idx_copy(0, 0).wait()
        for sl in range(W):
            gather(0, sl).start()
        idx_copy(1, 1).start()

        def group(g, carry):
            for slot in range(2):
                c = g * 2 + slot
                idx_copy(c + 1, 1 - slot).wait()
                for sl in range(W):
                    gather(1 - slot, sl).start()
                for sl in range(W):
                    gather(slot, sl).wait()
                @pl.when(c + 2 < nchunk)
                def _():
                    idx_copy(c + 2, slot).start()
                compute(c, slot)
            return carry

        lax.fori_loop(0, ngroup, group, 0)
        for sl in range(W):
            gather(0, sl).wait()
        compute(nchunk - 1, 0)
        slab_out(nchunk - 1, W - 2).wait()
        slab_out(nchunk - 1, W - 1).wait()

    return k


def kernel(token_ids, token_table, pos_table):
    b, seq = token_ids.shape
    vocab, d = token_table.shape
    ids_t = token_ids.T.astype(jnp.int32)
    pos_t = pos_table.T
    out5 = _emb_kernel(b, seq, d, vocab)(ids_t, token_table, pos_t)
    return out5.transpose(2, 4, 0, 1, 3).reshape(b, seq, d)


# speedup vs baseline: 1.5485x; 1.5485x over previous
import functools

import jax
import jax.numpy as jnp
from jax import lax
from jax.experimental import pallas as pl
from jax.experimental.pallas import tpu as pltpu
from jax.experimental.pallas import tpu_sc as plsc

NC = 2
NS = 16
NW = NC * NS
W = 8
LANE = 16
BT = 128
JU = 4


@functools.lru_cache(maxsize=None)
def _emb_kernel(b, seq, d, vocab):
    nchunk = seq // W
    ngroup = (nchunk - 1) // 2
    te = d // 8
    mesh = plsc.VectorSubcoreMesh(core_axis_name="c", subcore_axis_name="s")

    @functools.partial(
        pl.kernel,
        mesh=mesh,
        compiler_params=pltpu.CompilerParams(use_tc_tiling_on_sc=False,
                                             needs_layout_passes=False),
        out_type=jax.ShapeDtypeStruct((seq, te, b // BT, 8, BT), jnp.float32),
        scratch_types=[
            pltpu.VMEM((2, W, BT), jnp.int32),
            pltpu.VMEM((2, W, BT, d), jnp.float32),
            pltpu.VMEM((2, te, 8, BT + 1), jnp.float32),
            pltpu.VMEM((d, seq), jnp.float32),
            [pltpu.SemaphoreType.DMA] * 2,
            [pltpu.SemaphoreType.DMA] * 2,
            [pltpu.SemaphoreType.DMA] * 2,
        ],
    )
    def k(ids_t, tok_hbm, pos_t, out5, idx_v, rows_v, slab_v, pos_v,
          sem_i, sem_g, sem_o):
        wid = lax.axis_index("s") * NC + lax.axis_index("c")
        b0 = pl.multiple_of(wid * BT, BT)
        pltpu.sync_copy(pos_t.at[:, pl.ds(0, seq)], pos_v)
        iota = lax.iota(jnp.int32, 16)
        riota = [iota + h * LANE for h in range(d // LANE)]
        te_vec = [(iota + h * LANE) // 8 for h in range(d // LANE)]
        e8_vec = lax.rem(iota, 8)

        def idx_copy(c, slot):
            s0 = pl.multiple_of(c * W, W)
            return pltpu.make_async_copy(
                ids_t.at[pl.ds(s0, W), pl.ds(b0, BT)],
                idx_v.at[slot], sem_i[slot])

        def gather(slot, sl):
            return pltpu.make_async_copy(
                tok_hbm.at[idx_v.at[slot, sl]], rows_v.at[slot, sl],
                sem_g[slot])

        def slab_out(c, sl):
            return pltpu.make_async_copy(
                slab_v.at[sl % 2, :, :, pl.ds(0, BT)],
                out5.at[c * W + sl, :, wid], sem_o[sl % 2])

        def compute(c, slot):
            for sl in range(W):
                if sl >= 2:
                    slab_out(c, sl - 2).wait()
                else:
                    @pl.when(c >= 1)
                    def _():
                        slab_out(c - 1, W + sl - 2).wait()

                sg = jnp.full((16,), c * W + sl, jnp.int32)
                posrow = [plsc.load_gather(pos_v, [riota[h], sg])
                          for h in range(d // LANE)]

                def j_body(jo, carry):
                    for ju in range(JU):
                        j = jo * JU + ju
                        colj = jnp.full((16,), j, jnp.int32)
                        for h in range(d // LANE):
                            v = rows_v[slot, sl, j, pl.ds(h * LANE, LANE)]
                            plsc.store_scatter(
                                slab_v.at[sl % 2],
                                [te_vec[h], e8_vec, colj], v + posrow[h])
                    return carry

                lax.fori_loop(0, BT // JU, j_body, 0)
                slab_out(c, sl).start()

        idx_copy(0, 0).start()
        idx_copy(0, 0).wait()
        for sl in range(W):
            gather(0, sl).start()
        idx_copy(1, 1).start()

        def group(g, carry):
            for slot in range(2):
                c = g * 2 + slot
                idx_copy(c + 1, 1 - slot).wait()
                for sl in range(W):
                    gather(1 - slot, sl).start()
                for sl in range(W):
                    gather(slot, sl).wait()
                @pl.when(c + 2 < nchunk)
                def _():
                    idx_copy(c + 2, slot).start()
                compute(c, slot)
            return carry

        lax.fori_loop(0, ngroup, group, 0)
        for sl in range(W):
            gather(0, sl).wait()
        compute(nchunk - 1, 0)
        slab_out(nchunk - 1, W - 2).wait()
        slab_out(nchunk - 1, W - 1).wait()

    return k


def kernel(token_ids, token_table, pos_table):
    b, seq = token_ids.shape
    vocab, d = token_table.shape
    ids_t = token_ids.T.astype(jnp.int32)
    pos_t = pos_table.T
    out5 = _emb_kernel(b, seq, d, vocab)(ids_t, token_table, pos_t)
    return out5.transpose(2, 4, 0, 1, 3).reshape(b, seq, d)


# speedup vs baseline: 1.5512x; 1.0018x over previous
import functools

import jax
import jax.numpy as jnp
from jax import lax
from jax.experimental import pallas as pl
from jax.experimental.pallas import tpu as pltpu
from jax.experimental.pallas import tpu_sc as plsc

NC = 2
NS = 16
NW = NC * NS
W = 8
LANE = 16
BT = 128
JU = 8


@functools.lru_cache(maxsize=None)
def _emb_kernel(b, seq, d, vocab):
    nchunk = seq // W
    ngroup = (nchunk - 1) // 2
    te = d // 8
    mesh = plsc.VectorSubcoreMesh(core_axis_name="c", subcore_axis_name="s")

    @functools.partial(
        pl.kernel,
        mesh=mesh,
        compiler_params=pltpu.CompilerParams(use_tc_tiling_on_sc=False,
                                             needs_layout_passes=False),
        out_type=jax.ShapeDtypeStruct((seq, te, b // BT, 8, BT), jnp.float32),
        scratch_types=[
            pltpu.VMEM((2, W * BT), jnp.int32),
            pltpu.VMEM((2, W * BT, d), jnp.float32),
            pltpu.VMEM((2, te, 8, BT + 1), jnp.float32),
            pltpu.VMEM((d, seq), jnp.float32),
            [pltpu.SemaphoreType.DMA] * 2,
            [pltpu.SemaphoreType.DMA] * 2,
            [pltpu.SemaphoreType.DMA] * 2,
        ],
    )
    def k(ids_t, tok_hbm, pos_t, out5, idx_v, rows_v, slab_v, pos_v,
          sem_i, sem_g, sem_o):
        wid = lax.axis_index("s") * NC + lax.axis_index("c")
        b0 = pl.multiple_of(wid * BT, BT)
        pltpu.sync_copy(pos_t.at[:, pl.ds(0, seq)], pos_v)
        iota = lax.iota(jnp.int32, 16)
        riota = [iota + h * LANE for h in range(d // LANE)]
        te_vec = [(iota + h * LANE) // 8 for h in range(d // LANE)]
        e8_vec = lax.rem(iota, 8)

        def idx_copy(c, slot):
            i0 = pl.multiple_of(c * W * BT, W * BT)
            return pltpu.make_async_copy(
                ids_t.at[wid, pl.ds(i0, W * BT)],
                idx_v.at[slot], sem_i[slot])

        def gather(slot):
            return pltpu.make_async_copy(
                tok_hbm.at[idx_v.at[slot]], rows_v.at[slot], sem_g[slot])

        def slab_out(c, sl):
            return pltpu.make_async_copy(
                slab_v.at[sl % 2, :, :, pl.ds(0, BT)],
                out5.at[c * W + sl, :, wid], sem_o[sl % 2])

        def compute(c, slot):
            for sl in range(W):
                if sl >= 2:
                    slab_out(c, sl - 2).wait()
                else:
                    @pl.when(c >= 1)
                    def _():
                        slab_out(c - 1, W + sl - 2).wait()

                sg = jnp.full((16,), c * W + sl, jnp.int32)
                posrow = [plsc.load_gather(pos_v, [riota[h], sg])
                          for h in range(d // LANE)]

                def j_body(jo, carry):
                    for ju in range(JU):
                        j = jo * JU + ju
                        colj = jnp.full((16,), j, jnp.int32)
                        for h in range(d // LANE):
                            v = rows_v[slot, sl * BT + j,
                                       pl.ds(h * LANE, LANE)]
                            plsc.store_scatter(
                                slab_v.at[sl % 2],
                                [te_vec[h], e8_vec, colj], v + posrow[h])
                    return carry

                lax.fori_loop(0, BT // JU, j_body, 0)
                slab_out(c, sl).start()

        idx_copy(0, 0).start()
        idx_copy(0, 0).wait()
        gather(0).start()
        idx_copy(1, 1).start()

        def group(g, carry):
            for slot in range(2):
                c = g * 2 + slot
                idx_copy(c + 1, 1 - slot).wait()
                gather(1 - slot).start()
                gather(slot).wait()
                @pl.when(c + 2 < nchunk)
                def _():
                    idx_copy(c + 2, slot).start()
                compute(c, slot)
            return carry

        lax.fori_loop(0, ngroup, group, 0)
        gather(0).wait()
        compute(nchunk - 1, 0)
        slab_out(nchunk - 1, W - 2).wait()
        slab_out(nchunk - 1, W - 1).wait()

    return k


def kernel(token_ids, token_table, pos_table):
    b, seq = token_ids.shape
    vocab, d = token_table.shape
    ids_t = (token_ids.T.astype(jnp.int32)
             .reshape(seq, b // BT, BT)
             .transpose(1, 0, 2)
             .reshape(b // BT, seq * BT))
    pos_t = pos_table.T
    out5 = _emb_kernel(b, seq, d, vocab)(ids_t, token_table, pos_t)
    return out5.transpose(2, 4, 0, 1, 3).reshape(b, seq, d)
